# baseline (speedup 1.0000x reference)
import functools

import jax
import jax.numpy as jnp
from jax import lax
from jax.experimental import pallas as pl
from jax.experimental.pallas import tpu as pltpu
from jax.experimental.pallas import tpu_sc as plsc

VOCAB = 100000
DIM = 128
BATCH = 4096
HIST = 50

NC = 2
NS = 16
NW = NC * NS

TOTAL = BATCH * HIST
PER_W = TOTAL // NW
CHUNK = 320
NCHUNK = PER_W // CHUNK
NBUF = 2
NGROUP = NCHUNK // NBUF


def _emb_body(idx_hbm, table_hbm, out_hbm, idx_v, rows_v, gsem):
    wid = lax.axis_index("s") * NC + lax.axis_index("c")
    base = pl.multiple_of(wid * PER_W, PER_W)
    pltpu.sync_copy(idx_hbm.at[pl.ds(base, PER_W)], idx_v)

    def idx_slice(j):
        return idx_v.at[pl.ds(pl.multiple_of(j * CHUNK, CHUNK), CHUNK)]

    def start_gather(j, b):
        pltpu.async_copy(table_hbm.at[idx_slice(j)], rows_v.at[b], gsem.at[b])

    def finish_chunk(j, b):
        pltpu.make_async_copy(
            table_hbm.at[idx_slice(j)], rows_v.at[b], gsem.at[b]
        ).wait()
        out_row0 = pl.multiple_of(base + j * CHUNK, CHUNK)
        pltpu.sync_copy(rows_v.at[b], out_hbm.at[pl.ds(out_row0, CHUNK)])

    for b in range(NBUF):
        start_gather(b, b)

    def turn(g, carry):
        for b in range(NBUF):
            finish_chunk(g * NBUF + b, b)
            start_gather((g + 1) * NBUF + b, b)
        return carry

    lax.fori_loop(0, NGROUP - 1, turn, 0, unroll=False)

    for b in range(NBUF):
        finish_chunk((NGROUP - 1) * NBUF + b, b)


_mesh = plsc.VectorSubcoreMesh(core_axis_name="c", subcore_axis_name="s")

_emb = functools.partial(
    pl.kernel,
    mesh=_mesh,
    out_type=jax.ShapeDtypeStruct((TOTAL, DIM), jnp.float32),
    scratch_types=[
        pltpu.VMEM((PER_W,), jnp.int32),
        pltpu.VMEM((NBUF, CHUNK, DIM), jnp.float32),
        pltpu.SemaphoreType.DMA((NBUF,)),
    ],
)(_emb_body)


def kernel(indices, table):
    idx = indices.reshape(TOTAL).astype(jnp.int32)
    out = _emb(idx, table)
    return out.reshape(BATCH, HIST, DIM)


# speedup vs baseline: 1.7754x; 1.7754x over previous
import functools

import jax
import jax.numpy as jnp
from jax import lax
from jax.experimental import pallas as pl
from jax.experimental.pallas import tpu as pltpu
from jax.experimental.pallas import tpu_sc as plsc

VOCAB = 100000
DIM = 128
BATCH = 4096
HIST = 50
HIST_PAD = 64

NC = 2
NS = 16
NW = NC * NS

PER_B = BATCH // NW
KB = 8
NCHUNK = PER_B // KB
NBUF = 2
NGROUP = NCHUNK // NBUF


def _emb_body(idx_hbm, table_hbm, out_hbm, idx_v, rows_v, gsem):
    wid = lax.axis_index("s") * NC + lax.axis_index("c")
    b0 = pl.multiple_of(wid * PER_B, PER_B)
    pltpu.sync_copy(idx_hbm.at[pl.ds(b0, PER_B)], idx_v)

    def start_gathers(j, b):
        for kb in range(KB):
            pltpu.async_copy(
                table_hbm.at[idx_v.at[j * KB + kb, pl.ds(0, HIST)]],
                rows_v.at[b, kb],
                gsem.at[b],
            )

    def finish_chunk(j, b):
        for kb in range(KB):
            pltpu.make_async_copy(
                table_hbm.at[idx_v.at[j * KB + kb, pl.ds(0, HIST)]],
                rows_v.at[b, kb],
                gsem.at[b],
            ).wait()
        pltpu.sync_copy(rows_v.at[b], out_hbm.at[pl.ds(b0 + j * KB, KB)])

    for b in range(NBUF):
        start_gathers(b, b)

    def turn(g, carry):
        for b in range(NBUF):
            finish_chunk(g * NBUF + b, b)
            start_gathers((g + 1) * NBUF + b, b)
        return carry

    lax.fori_loop(0, NGROUP - 1, turn, 0, unroll=False)

    for b in range(NBUF):
        finish_chunk((NGROUP - 1) * NBUF + b, b)


_mesh = plsc.VectorSubcoreMesh(core_axis_name="c", subcore_axis_name="s")

_emb = functools.partial(
    pl.kernel,
    mesh=_mesh,
    out_type=jax.ShapeDtypeStruct((BATCH, HIST, DIM), jnp.float32),
    scratch_types=[
        pltpu.VMEM((PER_B, HIST_PAD), jnp.int32),
        pltpu.VMEM((NBUF, KB, HIST, DIM), jnp.float32),
        pltpu.SemaphoreType.DMA((NBUF,)),
    ],
)(_emb_body)


def kernel(indices, table):
    idx = jnp.pad(indices.astype(jnp.int32), ((0, 0), (0, HIST_PAD - HIST)))
    return _emb(idx, table)


# speedup vs baseline: 3.1948x; 1.7995x over previous
import functools

import jax
import jax.numpy as jnp
from jax import lax
from jax.experimental import pallas as pl
from jax.experimental.pallas import tpu as pltpu
from jax.experimental.pallas import tpu_sc as plsc

VOCAB = 100000
DIM = 128
BATCH = 4096
HIST = 50

NC = 2
NS = 16
NW = NC * NS

PER_B = BATCH // NW
NBUF = 5
NGROUP = HIST // NBUF


def _emb_body(idx_hbm, table_hbm, out_hbm, idx_v, rows_v, gsem):
    wid = lax.axis_index("s") * NC + lax.axis_index("c")
    b0 = pl.multiple_of(wid * PER_B, PER_B)
    pltpu.sync_copy(idx_hbm.at[:, pl.ds(b0, PER_B)], idx_v)

    def start_gather(h, b):
        pltpu.async_copy(table_hbm.at[idx_v.at[h]], rows_v.at[b], gsem.at[b])

    def finish_chunk(h, b):
        pltpu.make_async_copy(
            table_hbm.at[idx_v.at[h]], rows_v.at[b], gsem.at[b]
        ).wait()
        pltpu.sync_copy(rows_v.at[b], out_hbm.at[h, pl.ds(b0, PER_B)])

    for b in range(NBUF):
        start_gather(b, b)

    def turn(g, carry):
        for b in range(NBUF):
            finish_chunk(g * NBUF + b, b)
            start_gather((g + 1) * NBUF + b, b)
        return carry

    lax.fori_loop(0, NGROUP - 1, turn, 0, unroll=False)

    for b in range(NBUF):
        finish_chunk((NGROUP - 1) * NBUF + b, b)


_mesh = plsc.VectorSubcoreMesh(core_axis_name="c", subcore_axis_name="s")

_emb = functools.partial(
    pl.kernel,
    mesh=_mesh,
    out_type=jax.ShapeDtypeStruct((HIST, BATCH, DIM), jnp.float32),
    scratch_types=[
        pltpu.VMEM((HIST, PER_B), jnp.int32),
        pltpu.VMEM((NBUF, PER_B, DIM), jnp.float32),
        pltpu.SemaphoreType.DMA((NBUF,)),
    ],
)(_emb_body)


def kernel(indices, table):
    idx_t = jnp.transpose(indices.astype(jnp.int32))
    out = _emb(idx_t, table)
    return jnp.transpose(out, (1, 0, 2))


# speedup vs baseline: 3.2001x; 1.0017x over previous
import functools

import jax
import jax.numpy as jnp
from jax import lax
from jax.experimental import pallas as pl
from jax.experimental.pallas import tpu as pltpu
from jax.experimental.pallas import tpu_sc as plsc

VOCAB = 100000
DIM = 128
BATCH = 4096
HIST = 50

NC = 2
NS = 16
NW = NC * NS

PER_B = BATCH // NW
NBUF = 5
NGROUP = HIST // NBUF


GLEAD = 3
SLAG = NBUF - GLEAD


def _emb_body(idx_hbm, table_hbm, out_hbm, idx_v, rows_v, gsem, ssem):
    wid = lax.axis_index("s") * NC + lax.axis_index("c")
    b0 = pl.multiple_of(wid * PER_B, PER_B)
    pltpu.sync_copy(idx_hbm.at[:, pl.ds(b0, PER_B)], idx_v)

    def start_gather(h, b):
        pltpu.async_copy(table_hbm.at[idx_v.at[h]], rows_v.at[b], gsem.at[b])

    def wait_gather(h, b):
        pltpu.make_async_copy(
            table_hbm.at[idx_v.at[h]], rows_v.at[b], gsem.at[b]
        ).wait()

    def start_store(h, b):
        pltpu.async_copy(rows_v.at[b], out_hbm.at[h, pl.ds(b0, PER_B)], ssem.at[b])

    def wait_store(h, b):
        pltpu.make_async_copy(
            rows_v.at[b], out_hbm.at[h, pl.ds(b0, PER_B)], ssem.at[b]
        ).wait()

    for h in range(GLEAD):
        start_gather(h, h)

    def step(h, b):
        wait_gather(h, b)
        start_store(h, b)
        b2 = (b + GLEAD) % NBUF
        if isinstance(h, int) and h < SLAG:
            pass
        else:
            wait_store(h - SLAG, b2)
        if not (isinstance(h, int) and h + GLEAD >= HIST):
            start_gather(h + GLEAD, b2)

    for b in range(NBUF):
        step(b, b)

    def turn(g, carry):
        for b in range(NBUF):
            step(g * NBUF + b, b)
        return carry

    lax.fori_loop(1, NGROUP - 1, turn, 0, unroll=False)

    for b in range(NBUF):
        h = (NGROUP - 1) * NBUF + b
        wait_gather(h, b)
        start_store(h, b)
        if h + GLEAD < HIST:
            b2 = (b + GLEAD) % NBUF
            wait_store(h - SLAG, b2)
            start_gather(h + GLEAD, b2)

    for b in range(NBUF):
        h = HIST - NBUF + b
        wait_store(h, b)


_mesh = plsc.VectorSubcoreMesh(core_axis_name="c", subcore_axis_name="s")

_emb = functools.partial(
    pl.kernel,
    mesh=_mesh,
    out_type=jax.ShapeDtypeStruct((HIST, BATCH, DIM), jnp.float32),
    scratch_types=[
        pltpu.VMEM((HIST, PER_B), jnp.int32),
        pltpu.VMEM((NBUF, PER_B, DIM), jnp.float32),
        pltpu.SemaphoreType.DMA((NBUF,)),
        pltpu.SemaphoreType.DMA((NBUF,)),
    ],
)(_emb_body)


def kernel(indices, table):
    idx_t = jnp.transpose(indices.astype(jnp.int32))
    out = _emb(idx_t, table)
    return jnp.transpose(out, (1, 0, 2))
